# baseline (speedup 1.0000x reference)
import functools

import jax
import jax.numpy as jnp
from jax import lax
from jax.experimental import pallas as pl
from jax.experimental.pallas import tpu as pltpu
from jax.experimental.pallas import tpu_sc as plsc

NV = 10000
NE = 5000
NNZ = 320000
D = 128
NC, NS, L = 2, 16, 16
NW = NC * NS
P = NNZ // NW
K = 80
NBLK = P // K
NEp = 5008

_MESH = plsc.VectorSubcoreMesh(
    core_axis_name="c", subcore_axis_name="s", num_cores=NC, num_subcores=NS)


def _make_seg(T, with_counts=False, nbuf=4, ZC=40):
    NCH = T // ZC
    CH = 25
    NG = NBLK // CH
    CPG = P // L // NG
    CHK = CH * K
    HB = 3
    MI = (CH - 2 - HB) // nbuf

    out_type = jax.ShapeDtypeStruct((NC, T, D), jnp.float32)
    scratch = (
        [pltpu.VMEM((CHK,), jnp.int32) for _ in range(4)]
        + [pltpu.VMEM((K, D), jnp.float32) for _ in range(nbuf)]
        + [pltpu.VMEM((ZC, D), jnp.float32),
           pltpu.VMEM_SHARED((T, D), jnp.float32)]
        + [pltpu.SemaphoreType.DMA for _ in range(2 * nbuf + 1)]
    )
    if with_counts:
        out_type = [out_type,
                    jax.ShapeDtypeStruct((NW, NV), jnp.float32),
                    jax.ShapeDtypeStruct((NW, NEp), jnp.float32)]
        scratch += [
            pltpu.VMEM((P,), jnp.int32),
            pltpu.VMEM((P,), jnp.int32),
            pltpu.VMEM((NV,), jnp.float32),
            pltpu.VMEM((NEp,), jnp.float32),
        ]

    @functools.partial(
        pl.kernel, out_type=out_type, mesh=_MESH, scratch_types=scratch,
        compiler_params=pltpu.CompilerParams(needs_layout_passes=False),
    )
    def seg(*args):
        if with_counts:
            src, gid, sid, out, vout, eout = args[:6]
            rest = args[6:]
        else:
            src, gid, sid, out = args[:4]
            rest = args[4:]
        gidx = rest[0:2]
        sidx = rest[2:4]
        rows = rest[4:4 + nbuf]
        zbuf = rest[4 + nbuf]
        acc = rest[5 + nbuf]
        gsem = rest[6 + nbuf:6 + 2 * nbuf]
        ssem = rest[6 + 2 * nbuf:6 + 3 * nbuf]
        isem = rest[6 + 3 * nbuf]
        if with_counts:
            fvid, feid, vcnt, ecnt = rest[7 + 3 * nbuf:]
        cid = lax.axis_index("c")
        tid = lax.axis_index("s")
        wid = tid * NC + cid
        z = jnp.zeros((L,), jnp.float32)

        def zb(i, carry):
            zbuf[i // (D // L), pl.ds((i % (D // L)) * L, L)] = z
            return carry
        lax.fori_loop(0, ZC * D // L, zb, 0)
        base = wid * P
        pltpu.sync_copy(gid.at[pl.ds(base, CHK)], gidx[0])
        pltpu.sync_copy(sid.at[pl.ds(base, CHK)], sidx[0])
        pltpu.async_copy(src.at[gidx[0].at[pl.ds(0, K)]], rows[0], gsem[0])
        pltpu.async_copy(src.at[gidx[0].at[pl.ds(K, K)]], rows[1], gsem[1])

        def zc(i, carry):
            ch = tid + i * NS

            @pl.when(ch < NCH)
            def _():
                pltpu.sync_copy(zbuf, acc.at[pl.ds(ch * ZC, ZC)])
            return carry
        lax.fori_loop(0, -(-NCH // NS), zc, 0)
        if with_counts:
            pltpu.sync_copy(gid.at[pl.ds(base, P)], fvid)
            pltpu.sync_copy(sid.at[pl.ds(base, P)], feid)

            def zn(i, carry):
                vcnt[pl.ds(i * L, L)] = z
                return carry
            lax.fori_loop(0, NV // L, zn, 0)

            def zep(i, carry):
                ecnt[pl.ds(i * L, L)] = z
                return carry
            lax.fori_loop(0, NEp // L, zep, 0)
        plsc.subcore_barrier()

        def estep(a, l, u, w, swait, nxt):
            gi = gidx[a].at[pl.ds(l * K, K)]
            si = sidx[a].at[pl.ds(l * K, K)]
            pltpu.make_async_copy(src.at[gi], rows[u], gsem[u]).wait()
            pltpu.async_copy(rows[u], acc.at[si], ssem[u], add=True)
            if swait:
                pltpu.make_async_copy(rows[w], acc.at[si], ssem[w]).wait()
            if nxt is not None:
                s2, l2 = nxt
                pltpu.async_copy(
                    src.at[gidx[s2].at[pl.ds(l2 * K, K)]], rows[w], gsem[w])

        for g in range(NG):
            a, b = g % 2, (g + 1) % 2
            og = (g * CH) % nbuf
            if g + 1 < NG:
                pltpu.async_copy(
                    gid.at[pl.ds(base + (g + 1) * CHK, CHK)], gidx[b], isem)
                pltpu.async_copy(
                    sid.at[pl.ds(base + (g + 1) * CHK, CHK)], sidx[b], isem)
            if with_counts:
                ones = jnp.ones((L,), jnp.float32)

                def cnt(i, carry):
                    plsc.addupdate_scatter(
                        vcnt, [fvid[pl.ds(i * L, L)]], ones)
                    plsc.addupdate_scatter(
                        ecnt, [feid[pl.ds(i * L, L)]], ones)
                    return carry
                lax.fori_loop(g * CPG, (g + 1) * CPG, cnt, 0)
            for l in range(HB):
                estep(a, l, (og + l) % nbuf, (og + l + 2) % nbuf,
                      swait=(nbuf == 2 or g * CH + l >= 2), nxt=(a, l + 2))

            def body(jj, carry, a=a, og=og):
                for par in range(nbuf):
                    l = nbuf * jj + HB + par
                    estep(a, l, (og + HB + par) % nbuf,
                          (og + HB + par + 2) % nbuf, True, (a, l + 2))
                return carry
            lax.fori_loop(0, MI, body, 0)
            u2, w2 = (og + CH - 2) % nbuf, (og + CH) % nbuf
            u3, w3 = (og + CH - 1) % nbuf, (og + CH + 1) % nbuf
            if g + 1 < NG:
                pltpu.make_async_copy(
                    gid.at[pl.ds(base + (g + 1) * CHK, CHK)], gidx[b],
                    isem).wait()
                pltpu.make_async_copy(
                    sid.at[pl.ds(base + (g + 1) * CHK, CHK)], sidx[b],
                    isem).wait()
                estep(a, CH - 2, u2, w2, True, (b, 0))
                estep(a, CH - 1, u3, w3, True, (b, 1))
            else:
                estep(a, CH - 2, u2, w2, True, None)
                estep(a, CH - 1, u3, w3, True, None)
        if nbuf == 4:
            al = (NG - 1) % 2
            ol = ((NG - 1) * CH) % nbuf
            si0 = sidx[al].at[pl.ds(0, K)]
            pltpu.make_async_copy(
                rows[(ol + CH - 2) % nbuf], acc.at[si0],
                ssem[(ol + CH - 2) % nbuf]).wait()
            pltpu.make_async_copy(
                rows[(ol + CH - 1) % nbuf], acc.at[si0],
                ssem[(ol + CH - 1) % nbuf]).wait()
        if with_counts:
            pltpu.sync_copy(vcnt, vout.at[wid])
            pltpu.sync_copy(ecnt, eout.at[wid])
        plsc.subcore_barrier()

        @pl.when(tid == 0)
        def _():
            pltpu.sync_copy(acc.at[pl.ds(0, T)], out.at[cid])

    return seg


_SEG_E1 = _make_seg(NE, with_counts=True)
_SEG_E2 = _make_seg(NE)
_SEG_V = _make_seg(NV, ZC=8)


def _mm_kernel(x_ref, w_ref, b_ref, o_ref):
    o_ref[...] = jnp.dot(x_ref[...], w_ref[...],
                         preferred_element_type=jnp.float32) + b_ref[...]


def _mm(x, w, b, bn=1000):
    n = x.shape[0]
    return pl.pallas_call(
        _mm_kernel,
        grid=(n // bn,),
        in_specs=[
            pl.BlockSpec((bn, D), lambda i: (i, 0)),
            pl.BlockSpec((D, D), lambda i: (0, 0)),
            pl.BlockSpec((1, D), lambda i: (0, 0)),
        ],
        out_specs=pl.BlockSpec((bn, D), lambda i: (i, 0)),
        out_shape=jax.ShapeDtypeStruct((n, D), jnp.float32),
    )(x, w, b.reshape(1, D))


def _invprep_kernel(v_ref, e_ref, vi_ref, ei_ref):
    vi_ref[...] = (1.0 / jnp.maximum(
        jnp.sum(v_ref[...], axis=0), 1.0))[:, None]
    ei_ref[...] = (1.0 / jnp.maximum(
        jnp.sum(e_ref[...], axis=0), 1.0))[:, None]


def _invprep(vcnt_p, ecnt_p):
    return pl.pallas_call(
        _invprep_kernel,
        in_specs=[
            pl.BlockSpec((NW, NV), lambda: (0, 0)),
            pl.BlockSpec((NW, NEp), lambda: (0, 0)),
        ],
        out_specs=[
            pl.BlockSpec((NV, 1), lambda: (0, 0)),
            pl.BlockSpec((NEp, 1), lambda: (0, 0)),
        ],
        out_shape=[jax.ShapeDtypeStruct((NV, 1), jnp.float32),
                   jax.ShapeDtypeStruct((NEp, 1), jnp.float32)],
    )(vcnt_p, ecnt_p)


def _comb_kernel(p_ref, c_ref, o_ref):
    o_ref[...] = (p_ref[0] + p_ref[1]) * c_ref[...]


def _comb(parts, inv, bn=1000):
    t = parts.shape[1]
    return pl.pallas_call(
        _comb_kernel,
        grid=(t // bn,),
        in_specs=[
            pl.BlockSpec((NC, bn, D), lambda i: (0, i, 0)),
            pl.BlockSpec((bn, 1), lambda i: (i, 0)),
        ],
        out_specs=pl.BlockSpec((bn, D), lambda i: (i, 0)),
        out_shape=jax.ShapeDtypeStruct((t, D), jnp.float32),
    )(parts, inv)


def _comb_relu_mm_kernel(p_ref, c_ref, w_ref, b_ref, o_ref):
    x = jnp.maximum((p_ref[0] + p_ref[1]) * c_ref[...], 0.0)
    o_ref[...] = jnp.dot(x, w_ref[...],
                         preferred_element_type=jnp.float32) + b_ref[...]


def _comb_relu_mm(parts, inv, w, b, bn=1000):
    t = parts.shape[1]
    return pl.pallas_call(
        _comb_relu_mm_kernel,
        grid=(t // bn,),
        in_specs=[
            pl.BlockSpec((NC, bn, D), lambda i: (0, i, 0)),
            pl.BlockSpec((bn, 1), lambda i: (i, 0)),
            pl.BlockSpec((D, D), lambda i: (0, 0)),
            pl.BlockSpec((1, D), lambda i: (0, 0)),
        ],
        out_specs=pl.BlockSpec((bn, D), lambda i: (i, 0)),
        out_shape=jax.ShapeDtypeStruct((t, D), jnp.float32),
    )(parts, inv, w, b.reshape(1, D))


def kernel(X, v_ids, e_ids, W1, b1, W2, b2):
    y1 = _mm(X, W1, b1)
    e1, vcnt_p, ecnt_p = _SEG_E1(y1, v_ids, e_ids)
    vinv, einv = _invprep(vcnt_p, ecnt_p)
    einv = einv[:NE]
    he1 = _comb(e1, einv)
    v1 = _SEG_V(he1, e_ids, v_ids)
    x2 = _comb_relu_mm(v1, vinv, W2, b2)
    e2 = _SEG_E2(x2, v_ids, e_ids)
    he2 = _comb(e2, einv)
    v2 = _SEG_V(he2, e_ids, v_ids)
    return _comb(v2, vinv)


# speedup vs baseline: 1.0592x; 1.0592x over previous
import functools

import jax
import jax.numpy as jnp
from jax import lax
from jax.experimental import pallas as pl
from jax.experimental.pallas import tpu as pltpu
from jax.experimental.pallas import tpu_sc as plsc

NV = 10000
NE = 5000
NNZ = 320000
D = 128
NC, NS, L = 2, 16, 16
NW = NC * NS
P = NNZ // NW
K = 125
NBLK = P // K
NEp = 5008

_MESH = plsc.VectorSubcoreMesh(
    core_axis_name="c", subcore_axis_name="s", num_cores=NC, num_subcores=NS)


def _make_seg(T, with_counts=False, nbuf=2):
    ZC = 40
    NCH = T // ZC
    CH = 16
    NG = NBLK // CH
    CPG = P // L // NG

    out_type = jax.ShapeDtypeStruct((NC, T, D), jnp.float32)
    scratch = (
        [pltpu.VMEM((2, CH, K), jnp.int32),
         pltpu.VMEM((2, CH, K), jnp.int32)]
        + [pltpu.VMEM((K, D), jnp.float32) for _ in range(nbuf)]
        + [pltpu.VMEM((ZC, D), jnp.float32),
           pltpu.VMEM_SHARED((T, D), jnp.float32)]
        + [pltpu.SemaphoreType.DMA for _ in range(2 * nbuf + 1)]
    )
    if with_counts:
        out_type = [out_type,
                    jax.ShapeDtypeStruct((NW, NV), jnp.float32),
                    jax.ShapeDtypeStruct((NW, NEp), jnp.float32)]
        scratch += [
            pltpu.VMEM((P,), jnp.int32),
            pltpu.VMEM((P,), jnp.int32),
            pltpu.VMEM((NV,), jnp.float32),
            pltpu.VMEM((NEp,), jnp.float32),
        ]

    @functools.partial(
        pl.kernel, out_type=out_type, mesh=_MESH, scratch_types=scratch,
        compiler_params=pltpu.CompilerParams(needs_layout_passes=False),
    )
    def seg(*args):
        if with_counts:
            src, gid, sid, vidf, eidf, out, vout, eout = args[:8]
            rest = args[8:]
        else:
            src, gid, sid, out = args[:4]
            rest = args[4:]
        gidx, sidx = rest[0], rest[1]
        rows = rest[2:2 + nbuf]
        zbuf = rest[2 + nbuf]
        acc = rest[3 + nbuf]
        gsem = rest[4 + nbuf:4 + 2 * nbuf]
        ssem = rest[4 + 2 * nbuf:4 + 3 * nbuf]
        isem = rest[4 + 3 * nbuf]
        if with_counts:
            fvid, feid, vcnt, ecnt = rest[5 + 3 * nbuf:]
        cid = lax.axis_index("c")
        tid = lax.axis_index("s")
        wid = tid * NC + cid
        z = jnp.zeros((L,), jnp.float32)

        def zb(i, carry):
            zbuf[i // (D // L), pl.ds((i % (D // L)) * L, L)] = z
            return carry
        lax.fori_loop(0, ZC * D // L, zb, 0)
        pltpu.sync_copy(gid.at[wid, pl.ds(0, CH)], gidx.at[0])
        pltpu.sync_copy(sid.at[wid, pl.ds(0, CH)], sidx.at[0])
        pltpu.async_copy(src.at[gidx.at[0, 0]], rows[0], gsem[0])
        pltpu.async_copy(src.at[gidx.at[0, 1]], rows[1], gsem[1])

        def zc(i, carry):
            ch = tid + i * NS

            @pl.when(ch < NCH)
            def _():
                pltpu.sync_copy(zbuf, acc.at[pl.ds(ch * ZC, ZC)])
            return carry
        lax.fori_loop(0, -(-NCH // NS), zc, 0)
        if with_counts:
            pltpu.sync_copy(vidf.at[pl.ds(wid * P, P)], fvid)
            pltpu.sync_copy(eidf.at[pl.ds(wid * P, P)], feid)

            def zn(i, carry):
                vcnt[pl.ds(i * L, L)] = z
                return carry
            lax.fori_loop(0, NV // L, zn, 0)

            def zep(i, carry):
                ecnt[pl.ds(i * L, L)] = z
                return carry
            lax.fori_loop(0, NEp // L, zep, 0)
        plsc.subcore_barrier()

        def estep(a, l, u, w, swait, nxt):
            pltpu.make_async_copy(
                src.at[gidx.at[a, l]], rows[u], gsem[u]).wait()
            pltpu.async_copy(rows[u], acc.at[sidx.at[a, l]], ssem[u],
                             add=True)
            if swait:
                pltpu.make_async_copy(
                    rows[w], acc.at[sidx.at[a, l]], ssem[w]).wait()
            if nxt is not None:
                s2, l2 = nxt
                pltpu.async_copy(src.at[gidx.at[s2, l2]], rows[w], gsem[w])

        for g in range(NG):
            a, b = g % 2, (g + 1) % 2
            if g + 1 < NG:
                pltpu.async_copy(
                    gid.at[wid, pl.ds((g + 1) * CH, CH)], gidx.at[b], isem)
                pltpu.async_copy(
                    sid.at[wid, pl.ds((g + 1) * CH, CH)], sidx.at[b], isem)
            if with_counts:
                ones = jnp.ones((L,), jnp.float32)

                def cnt(i, carry):
                    plsc.addupdate_scatter(
                        vcnt, [fvid[pl.ds(i * L, L)]], ones)
                    plsc.addupdate_scatter(
                        ecnt, [feid[pl.ds(i * L, L)]], ones)
                    return carry
                lax.fori_loop(g * CPG, (g + 1) * CPG, cnt, 0)
            for l in (0, 1):
                estep(a, l, l % nbuf, (l + 2) % nbuf,
                      swait=(g > 0 or nbuf == 2), nxt=(a, l + 2))

            def body(jj, carry, a=a):
                for par in range(nbuf):
                    l = nbuf * jj + 2 + par
                    estep(a, l, (2 + par) % nbuf, (4 + par) % nbuf,
                          True, (a, l + 2))
                return carry
            lax.fori_loop(0, (CH - 4) // nbuf, body, 0)
            if g + 1 < NG:
                pltpu.make_async_copy(
                    gid.at[wid, pl.ds((g + 1) * CH, CH)], gidx.at[b],
                    isem).wait()
                pltpu.make_async_copy(
                    sid.at[wid, pl.ds((g + 1) * CH, CH)], sidx.at[b],
                    isem).wait()
                estep(a, CH - 2, (CH - 2) % nbuf, CH % nbuf, True, (b, 0))
                estep(a, CH - 1, (CH - 1) % nbuf, (CH + 1) % nbuf, True,
                      (b, 1))
            else:
                estep(a, CH - 2, (CH - 2) % nbuf, CH % nbuf, True, None)
                estep(a, CH - 1, (CH - 1) % nbuf, (CH + 1) % nbuf, True,
                      None)
        if nbuf == 4:
            al = (NG - 1) % 2
            pltpu.make_async_copy(
                rows[(CH - 2) % nbuf], acc.at[sidx.at[al, 0]],
                ssem[(CH - 2) % nbuf]).wait()
            pltpu.make_async_copy(
                rows[(CH - 1) % nbuf], acc.at[sidx.at[al, 0]],
                ssem[(CH - 1) % nbuf]).wait()
        if with_counts:
            pltpu.sync_copy(vcnt, vout.at[wid])
            pltpu.sync_copy(ecnt, eout.at[wid])
        plsc.subcore_barrier()

        @pl.when(tid == 0)
        def _():
            pltpu.sync_copy(acc.at[pl.ds(0, T)], out.at[cid])

    return seg


_SEG_E1 = _make_seg(NE, with_counts=True)
_SEG_E2 = _make_seg(NE, nbuf=4)
_SEG_V = _make_seg(NV)


def _mm_kernel(x_ref, w_ref, b_ref, o_ref):
    o_ref[...] = jnp.dot(x_ref[...], w_ref[...],
                         preferred_element_type=jnp.float32) + b_ref[...]


def _mm(x, w, b, bn=1000):
    n = x.shape[0]
    return pl.pallas_call(
        _mm_kernel,
        grid=(n // bn,),
        in_specs=[
            pl.BlockSpec((bn, D), lambda i: (i, 0)),
            pl.BlockSpec((D, D), lambda i: (0, 0)),
            pl.BlockSpec((1, D), lambda i: (0, 0)),
        ],
        out_specs=pl.BlockSpec((bn, D), lambda i: (i, 0)),
        out_shape=jax.ShapeDtypeStruct((n, D), jnp.float32),
    )(x, w, b.reshape(1, D))


def _invprep_kernel(v_ref, e_ref, vi_ref, ei_ref):
    vi_ref[...] = (1.0 / jnp.maximum(
        jnp.sum(v_ref[...], axis=0), 1.0))[:, None]
    ei_ref[...] = (1.0 / jnp.maximum(
        jnp.sum(e_ref[...], axis=0), 1.0))[:NE, None]


def _invprep(vcnt_p, ecnt_p):
    return pl.pallas_call(
        _invprep_kernel,
        in_specs=[
            pl.BlockSpec((NW, NV), lambda: (0, 0)),
            pl.BlockSpec((NW, NEp), lambda: (0, 0)),
        ],
        out_specs=[
            pl.BlockSpec((NV, 1), lambda: (0, 0)),
            pl.BlockSpec((NE, 1), lambda: (0, 0)),
        ],
        out_shape=[jax.ShapeDtypeStruct((NV, 1), jnp.float32),
                   jax.ShapeDtypeStruct((NE, 1), jnp.float32)],
    )(vcnt_p, ecnt_p)


def _comb_kernel(p_ref, c_ref, o_ref):
    o_ref[...] = (p_ref[0] + p_ref[1]) * c_ref[...]


def _comb(parts, inv, bn=1000):
    t = parts.shape[1]
    return pl.pallas_call(
        _comb_kernel,
        grid=(t // bn,),
        in_specs=[
            pl.BlockSpec((NC, bn, D), lambda i: (0, i, 0)),
            pl.BlockSpec((bn, 1), lambda i: (i, 0)),
        ],
        out_specs=pl.BlockSpec((bn, D), lambda i: (i, 0)),
        out_shape=jax.ShapeDtypeStruct((t, D), jnp.float32),
    )(parts, inv)


def _comb_relu_mm_kernel(p_ref, c_ref, w_ref, b_ref, o_ref):
    x = jnp.maximum((p_ref[0] + p_ref[1]) * c_ref[...], 0.0)
    o_ref[...] = jnp.dot(x, w_ref[...],
                         preferred_element_type=jnp.float32) + b_ref[...]


def _comb_relu_mm(parts, inv, w, b, bn=1000):
    t = parts.shape[1]
    return pl.pallas_call(
        _comb_relu_mm_kernel,
        grid=(t // bn,),
        in_specs=[
            pl.BlockSpec((NC, bn, D), lambda i: (0, i, 0)),
            pl.BlockSpec((bn, 1), lambda i: (i, 0)),
            pl.BlockSpec((D, D), lambda i: (0, 0)),
            pl.BlockSpec((1, D), lambda i: (0, 0)),
        ],
        out_specs=pl.BlockSpec((bn, D), lambda i: (i, 0)),
        out_shape=jax.ShapeDtypeStruct((t, D), jnp.float32),
    )(parts, inv, w, b.reshape(1, D))


def kernel(X, v_ids, e_ids, W1, b1, W2, b2):
    gv = v_ids.reshape(NW, NBLK, K)
    ge = e_ids.reshape(NW, NBLK, K)

    y1 = _mm(X, W1, b1)
    e1, vcnt_p, ecnt_p = _SEG_E1(y1, gv, ge, v_ids, e_ids)
    vinv, einv = _invprep(vcnt_p, ecnt_p)
    he1 = _comb(e1, einv)
    v1 = _SEG_V(he1, ge, gv)
    x2 = _comb_relu_mm(v1, vinv, W2, b2)
    e2 = _SEG_E2(x2, gv, ge)
    he2 = _comb(e2, einv)
    v2 = _SEG_V(he2, ge, gv)
    return _comb(v2, vinv)
